# baseline (speedup 1.0000x reference)
import functools

import jax
import jax.numpy as jnp
from jax import lax
from jax.experimental import pallas as pl
from jax.experimental.pallas import tpu as pltpu
from jax.experimental.pallas import tpu_sc as plsc

_TCH = 384
_VMAIN = 999936


def _iota16():
    return lax.iota(jnp.int32, 16)


@jax.jit
def _run(tt, tail2, it):
    info = plsc.get_sparse_core_info()
    nc = info.num_cores
    mesh = plsc.VectorSubcoreMesh(core_axis_name="c", subcore_axis_name="s")
    nchunks = _VMAIN // _TCH
    trips = nchunks // 32 + 1

    k1_scratch = (
        [pltpu.VMEM((64, _TCH + 1), jnp.float32) for _ in range(2)]
        + [pltpu.VMEM((_TCH // 2, 128), jnp.float32) for _ in range(2)]
        + [pltpu.VMEM((32, 128), jnp.float32)]
        + [pltpu.SemaphoreType.DMA for _ in range(4)]
    )

    @functools.partial(
        pl.kernel,
        mesh=mesh,
        out_type=jax.ShapeDtypeStruct((500000, 128), jnp.float32),
        scratch_types=k1_scratch,
        compiler_params=pltpu.CompilerParams(needs_layout_passes=False),
    )
    def k1(tt_hbm, tail_hbm, tlin_hbm, vin0, vin1, vo0, vo1, vtail, g0, g1,
           s0, s1):
        vin = (vin0, vin1)
        vout = (vo0, vo1)
        gsem = (g0, g1)
        ssem = (s0, s1)
        wid = lax.axis_index("s") * nc + lax.axis_index("c")

        @pl.when(wid == 0)
        def _():
            pltpu.sync_copy(tail_hbm, vtail)
            pltpu.sync_copy(vtail, tlin_hbm.at[pl.ds(499968, 32)])

        rows16 = [_iota16() + 16 * mm for mm in range(4)]

        def in_start(g, b):
            pltpu.async_copy(tt_hbm.at[:, pl.ds(g * _TCH, _TCH)],
                             vin[b].at[:, pl.ds(0, _TCH)], gsem[b])

        def in_wait(b):
            pltpu.make_async_copy(tt_hbm.at[:, pl.ds(0, _TCH)],
                                  vin[b].at[:, pl.ds(0, _TCH)],
                                  gsem[b]).wait()

        def out_start(g, b):
            pltpu.async_copy(vout[b],
                             tlin_hbm.at[pl.ds(g * (_TCH // 2), _TCH // 2)],
                             ssem[b])

        def out_wait(b):
            pltpu.make_async_copy(vout[b],
                                  tlin_hbm.at[pl.ds(0, _TCH // 2)],
                                  ssem[b]).wait()

        def transpose(b):
            @plsc.parallel_loop(0, _TCH // 2, unroll=16)
            def _(k):
                c0 = jnp.broadcast_to(2 * k, (16,))
                c1 = c0 + 1
                for m in range(8):
                    cols = c0 if m < 4 else c1
                    vals = plsc.load_gather(vin[b], [rows16[m % 4], cols])
                    vout[b][k, pl.ds(16 * m, 16)] = vals

        in_start(wid, 0)

        def loop(i2, carry):
            for b in range(2):
                i = 2 * i2 + b
                g = wid + 32 * i

                @pl.when(g < nchunks)
                def _():
                    gn = g + 32

                    @pl.when(gn < nchunks)
                    def _():
                        in_start(gn, 1 - b)

                    in_wait(b)

                    @pl.when(i >= 2)
                    def _():
                        out_wait(b)

                    transpose(b)
                    out_start(g, b)

            return carry

        lax.fori_loop(0, (trips + 1) // 2, loop, 0)
        for b in range(2):
            out_wait(b)

    tlin = k1(tt, tail2)

    k2_scratch = (
        [pltpu.VMEM((8, 128), jnp.int32) for _ in range(2)]
        + [pltpu.VMEM((1024,), jnp.int32)]
        + [pltpu.VMEM((256, 128), jnp.float32) for _ in range(2)]
        + [pltpu.VMEM((2, 64, 128), jnp.float32) for _ in range(2)]
        + [pltpu.SemaphoreType.DMA for _ in range(4)]
    )

    @functools.partial(
        pl.kernel,
        mesh=mesh,
        out_type=jax.ShapeDtypeStruct((200, 64, 4096), jnp.float32),
        scratch_types=k2_scratch,
        compiler_params=pltpu.CompilerParams(needs_layout_passes=False),
    )
    def k2(tlin_hbm, it_hbm, op_hbm, iv, pb, jb, buf0, buf1, ob0, ob1,
           g0, g1, s0, s1):
        buf = (buf0, buf1)
        obuf = (ob0, ob1)
        gsem = (g0, g1)
        ssem = (s0, s1)
        wid = lax.axis_index("s") * nc + lax.axis_index("c")

        rowbase = [_iota16() + 16 * m for m in range(8)]

        def gather_start(si2, b):
            pltpu.async_copy(tlin_hbm.at[jb.at[pl.ds(256 * si2, 256)]],
                             buf[b], gsem[b])

        def gather_wait(b):
            pltpu.make_async_copy(tlin_hbm.at[jb.at[pl.ds(0, 256)]], buf[b],
                                  gsem[b]).wait()

        def out_start(sb, si2, b):
            pltpu.async_copy(
                obuf[b], op_hbm.at[pl.ds(8 * sb + 2 * si2, 2), :,
                                   pl.ds(128 * wid, 128)], ssem[b])

        def out_wait(b):
            pltpu.make_async_copy(
                obuf[b], op_hbm.at[pl.ds(0, 2), :, pl.ds(0, 128)],
                ssem[b]).wait()

        def prep_block(sb):
            pltpu.sync_copy(it_hbm.at[pl.ds(8 * sb, 8), pl.ds(128 * wid, 128)],
                            iv)
            for r in range(8):
                for m in range(8):
                    v = iv[r, pl.ds(16 * m, 16)]
                    jb[pl.ds(r * 128 + 16 * m, 16)] = v >> 1
                    pb[r, pl.ds(16 * m, 16)] = (v & 1) << 6

        def transpose_chunk(si2, b):
            pbs = []
            for si in range(2):
                for m in range(8):
                    pbs.append((si, m,
                                pb[2 * si2 + si, pl.ds(16 * m, 16)],
                                rowbase[m] + si * 128))

            @plsc.parallel_loop(0, 64, unroll=8)
            def _(e):
                for si, m, pvec, rvec in pbs:
                    vals = plsc.load_gather(buf[b], [rvec, pvec + e])
                    obuf[b][si, e, pl.ds(16 * m, 16)] = vals

        def block_loop(sb, carry):
            prep_block(sb)
            gather_start(0, 0)
            for si2 in range(4):
                b = si2 % 2
                if si2 < 3:
                    gather_start(si2 + 1, 1 - b)
                gather_wait(b)
                step = sb * 4 + si2

                @pl.when(step >= 2)
                def _():
                    out_wait(b)

                transpose_chunk(si2, b)
                out_start(sb, si2, b)
            return carry

        lax.fori_loop(0, 25, block_loop, 0)
        for b in range(2):
            out_wait(b)

    op = k2(tlin, it)
    return op


def kernel(indices, table):
    tt = table.T
    tail2 = lax.slice(table, (999936, 0), (1000000, 64)).reshape(32, 128)
    it = indices.T
    op = _run(tt, tail2, it)
    return op.transpose(2, 0, 1)


# speedup vs baseline: 1.3957x; 1.3957x over previous
import functools

import jax
import jax.numpy as jnp
from jax import lax
from jax.experimental import pallas as pl
from jax.experimental.pallas import tpu as pltpu
from jax.experimental.pallas import tpu_sc as plsc

_CHUNK = 400
_NBUF = 4


@jax.jit
def _gather_rows(idx_grouped, table):
    nw, nchunks, _ = idx_grouped.shape
    n = nw * nchunks * _CHUNK
    d = table.shape[1]
    per_worker = nchunks * _CHUNK
    nouter = nchunks // _NBUF
    info = plsc.get_sparse_core_info()
    assert nw == info.num_cores * info.num_subcores
    mesh = plsc.VectorSubcoreMesh(core_axis_name="c", subcore_axis_name="s")

    scratch = (
        [pltpu.VMEM((nchunks, _CHUNK), jnp.int32)]
        + [pltpu.VMEM((_CHUNK, d), jnp.float32) for _ in range(_NBUF)]
        + [pltpu.SemaphoreType.DMA for _ in range(2 * _NBUF)]
    )

    @functools.partial(
        pl.kernel,
        mesh=mesh,
        out_type=jax.ShapeDtypeStruct((n, d), jnp.float32),
        scratch_types=scratch,
        compiler_params=pltpu.CompilerParams(use_tc_tiling_on_sc=False),
    )
    def k(idx_hbm, table_hbm, out_hbm, idx_v, *bufs_and_sems):
        rows = bufs_and_sems[:_NBUF]
        gsem = bufs_and_sems[_NBUF:2 * _NBUF]
        ssem = bufs_and_sems[2 * _NBUF:]
        wid = lax.axis_index("s") * info.num_cores + lax.axis_index("c")
        base0 = wid * per_worker

        pltpu.sync_copy(idx_hbm.at[wid], idx_v)

        def gather_start(c, b):
            pltpu.async_copy(table_hbm.at[idx_v.at[c]], rows[b], gsem[b])

        def gather_wait(b):
            pltpu.make_async_copy(table_hbm.at[idx_v.at[0]], rows[b],
                                  gsem[b]).wait()

        def store_start(c, b):
            pltpu.async_copy(rows[b], out_hbm.at[pl.ds(base0 + c * _CHUNK,
                                                       _CHUNK)], ssem[b])

        def store_wait(b):
            pltpu.make_async_copy(rows[b], out_hbm.at[pl.ds(base0, _CHUNK)],
                                  ssem[b]).wait()

        for b in range(_NBUF):
            gather_start(b, b)

        def body(g, carry):
            for b in range(_NBUF):
                c = g * _NBUF + b
                gather_wait(b)
                store_start(c, b)
                nxt = c + _NBUF

                @pl.when(nxt < nchunks)
                def _():
                    store_wait(b)
                    gather_start(nxt, b)

            return carry

        lax.fori_loop(0, nouter, body, 0)
        for b in range(_NBUF):
            store_wait(b)

    return k(idx_grouped, table)


def kernel(indices, table):
    b, s = indices.shape
    d = table.shape[1]
    n = b * s
    info = plsc.get_sparse_core_info()
    nw = info.num_cores * info.num_subcores
    idx_grouped = indices.reshape(nw, n // (nw * _CHUNK), _CHUNK)
    out = _gather_rows(idx_grouped, table)
    return out.reshape(b, s, d)
